# baseline (speedup 1.0000x reference)
import dataclasses
import functools

import jax
import jax.numpy as jnp
from jax import lax
from jax.experimental import pallas as pl
from jax.experimental.pallas import tpu as pltpu
from jax.experimental.pallas import tpu_sc as plsc

BATCH = 16384
NUM_AREA = 100000
EMBED_DIM = 64
OUT_W = 4 * EMBED_DIM
ROW = 2 * EMBED_DIM
NC = 2
NS = 16
NW = NC * NS
B_PER_W = BATCH // NW
CHUNK = 128
NCH = B_PER_W // CHUNK
LANES = 16


_PV_BLK = 2000


def _pair_view(W_area):
    def body(x_ref, o_ref):
        x = x_ref[...].reshape(_PV_BLK // 2, 2, EMBED_DIM)
        o_ref[...] = jnp.concatenate([x[:, 0, :], x[:, 1, :]], axis=1)

    return pl.pallas_call(
        body,
        grid=(NUM_AREA // _PV_BLK,),
        in_specs=[pl.BlockSpec((_PV_BLK, EMBED_DIM), lambda i: (i, 0))],
        out_specs=pl.BlockSpec((_PV_BLK // 2, ROW), lambda i: (i, 0)),
        out_shape=jax.ShapeDtypeStruct((NUM_AREA // 2, ROW), jnp.float32),
    )(W_area)


def kernel(gender_idx, age_idx, occupation_idx, area_idx, u_id,
           W_gender, W_age, W_occupation, W_area):
    del u_id
    g = gender_idx.astype(jnp.int32)
    a = age_idx.astype(jnp.int32)
    o = occupation_idx.astype(jnp.int32)
    ar = area_idx.astype(jnp.int32)

    n_gao = 2 * 7 * 21
    cid = jnp.arange(n_gao, dtype=jnp.int32)
    W_gao = jnp.concatenate(
        [W_gender[cid // (7 * 21)], W_age[(cid // 21) % 7],
         W_occupation[cid % 21],
         jnp.zeros((n_gao, EMBED_DIM), jnp.float32)], axis=1)
    X_area = _pair_view(W_area)

    mesh = plsc.VectorSubcoreMesh(core_axis_name="c", subcore_axis_name="s")
    cp = pltpu.CompilerParams()
    if "needs_layout_passes" in pltpu.CompilerParams.__dataclass_fields__:
        cp = dataclasses.replace(cp, needs_layout_passes=False)

    @functools.partial(
        pl.kernel,
        mesh=mesh,
        compiler_params=cp,
        out_type=jax.ShapeDtypeStruct((BATCH, OUT_W), jnp.float32),
        scratch_types=[
            pltpu.VMEM((B_PER_W,), jnp.int32),
            pltpu.VMEM((B_PER_W,), jnp.int32),
            pltpu.VMEM((B_PER_W,), jnp.int32),
            pltpu.VMEM((B_PER_W,), jnp.int32),
            pltpu.VMEM((CHUNK, OUT_W), jnp.float32),
            pltpu.VMEM((CHUNK, OUT_W), jnp.float32),
            pltpu.VMEM((CHUNK, ROW), jnp.float32),
            pltpu.VMEM((CHUNK, ROW), jnp.float32),
            pltpu.SemaphoreType.DMA,
            pltpu.SemaphoreType.DMA,
            pltpu.SemaphoreType.DMA,
            pltpu.SemaphoreType.DMA,
            pltpu.SemaphoreType.DMA,
        ],
    )
    def emb_kernel(g_hbm, a_hbm, o_hbm, ar_hbm, wgao_hbm, xar_hbm, out_hbm,
                   gaov, tv, ov, hv, cat0, cat1, area0, area1,
                   semi, semg0, semg1, sems0, sems1):
        cats = (cat0, cat1)
        areas = (area0, area1)
        semg = (semg0, semg1)
        sems = (sems0, sems1)
        wid = lax.axis_index("s") * NC + lax.axis_index("c")
        base = wid * B_PER_W
        iot = lax.iota(jnp.int32, LANES)

        loads = [pltpu.async_copy(src.at[pl.ds(base, B_PER_W)], dst, semi)
                 for src, dst in ((g_hbm, gaov), (a_hbm, tv),
                                 (o_hbm, ov), (ar_hbm, hv))]
        for h in loads:
            h.wait()
        for t in range(B_PER_W // LANES):
            s = pl.ds(t * LANES, LANES)
            gaov.at[s][...] = (gaov.at[s][...] * 7 + tv.at[s][...]) * 21 + \
                ov.at[s][...]
            ar16 = hv.at[s][...]
            tv.at[s][...] = lax.shift_right_logical(ar16, 1)
            hv.at[s][...] = lax.shift_left(jnp.bitwise_and(ar16, 1), 6)

        def issue_gathers(c):
            s = c % 2
            off = pl.ds(c * CHUNK, CHUNK)
            return [
                pltpu.async_copy(wgao_hbm.at[gaov.at[off]], cats[s], semg[s]),
                pltpu.async_copy(xar_hbm.at[tv.at[off]], areas[s], semg[s]),
            ]

        def select_half(c):
            s = c % 2
            @pl.loop(0, CHUNK, step=LANES)
            def _(j0):
                rowv = iot + j0
                hvv = hv.at[pl.ds(c * CHUNK + j0, LANES)][...]
                colv = iot * 0 + (3 * EMBED_DIM)
                for cc in range(EMBED_DIM):
                    vals = plsc.load_gather(areas[s], [rowv, hvv + cc])
                    plsc.store_scatter(cats[s], [rowv, colv + cc], vals)

        pend_g = {0: issue_gathers(0), 1: None}
        pend_s = {0: None, 1: None}
        for c in range(NCH):
            s = c % 2
            for h in pend_g[s]:
                h.wait()
            if c + 1 < NCH:
                if pend_s[1 - s] is not None:
                    pend_s[1 - s].wait()
                    pend_s[1 - s] = None
                pend_g[1 - s] = issue_gathers(c + 1)
            select_half(c)
            pend_s[s] = pltpu.async_copy(
                cats[s], out_hbm.at[pl.ds(base + c * CHUNK, CHUNK)], sems[s])
        for s in (0, 1):
            if pend_s[s] is not None:
                pend_s[s].wait()

    return emb_kernel(g, a, o, ar, W_gao, X_area)


# speedup vs baseline: 1.2471x; 1.2471x over previous
import dataclasses
import functools

import jax
import jax.numpy as jnp
from jax import lax
from jax.experimental import pallas as pl
from jax.experimental.pallas import tpu as pltpu
from jax.experimental.pallas import tpu_sc as plsc

BATCH = 16384
NUM_AREA = 100000
EMBED_DIM = 64
OUT_W = 4 * EMBED_DIM
ROW = 2 * EMBED_DIM
NC = 2
NS = 16
NW = NC * NS
B_PER_W = BATCH // NW
CHUNK = 128
NCH = B_PER_W // CHUNK
LANES = 16


def kernel(gender_idx, age_idx, occupation_idx, area_idx, u_id,
           W_gender, W_age, W_occupation, W_area):
    del u_id
    g = gender_idx.astype(jnp.int32)
    a = age_idx.astype(jnp.int32)
    o = occupation_idx.astype(jnp.int32)
    ar = area_idx.astype(jnp.int32)

    n_gao = 2 * 7 * 21
    cid = jnp.arange(n_gao, dtype=jnp.int32)
    W_gao = jnp.concatenate(
        [W_gender[cid // (7 * 21)], W_age[(cid // 21) % 7],
         W_occupation[cid % 21],
         jnp.zeros((n_gao, EMBED_DIM), jnp.float32)], axis=1)
    X_area = W_area.reshape(-1, ROW)

    mesh = plsc.VectorSubcoreMesh(core_axis_name="c", subcore_axis_name="s")
    cp = pltpu.CompilerParams()
    if "needs_layout_passes" in pltpu.CompilerParams.__dataclass_fields__:
        cp = dataclasses.replace(cp, needs_layout_passes=False)

    @functools.partial(
        pl.kernel,
        mesh=mesh,
        compiler_params=cp,
        out_type=jax.ShapeDtypeStruct((BATCH, OUT_W), jnp.float32),
        scratch_types=[
            pltpu.VMEM((B_PER_W,), jnp.int32),
            pltpu.VMEM((B_PER_W,), jnp.int32),
            pltpu.VMEM((B_PER_W,), jnp.int32),
            pltpu.VMEM((B_PER_W,), jnp.int32),
            pltpu.VMEM((CHUNK, OUT_W), jnp.float32),
            pltpu.VMEM((CHUNK, OUT_W), jnp.float32),
            pltpu.VMEM((CHUNK, ROW), jnp.float32),
            pltpu.VMEM((CHUNK, ROW), jnp.float32),
            pltpu.SemaphoreType.DMA,
            pltpu.SemaphoreType.DMA,
            pltpu.SemaphoreType.DMA,
            pltpu.SemaphoreType.DMA,
            pltpu.SemaphoreType.DMA,
        ],
    )
    def emb_kernel(g_hbm, a_hbm, o_hbm, ar_hbm, wgao_hbm, xar_hbm, out_hbm,
                   gaov, tv, ov, hv, cat0, cat1, area0, area1,
                   semi, semg0, semg1, sems0, sems1):
        cats = (cat0, cat1)
        areas = (area0, area1)
        semg = (semg0, semg1)
        sems = (sems0, sems1)
        wid = lax.axis_index("s") * NC + lax.axis_index("c")
        base = wid * B_PER_W
        iot = lax.iota(jnp.int32, LANES)

        loads = [pltpu.async_copy(src.at[pl.ds(base, B_PER_W)], dst, semi)
                 for src, dst in ((g_hbm, gaov), (a_hbm, tv),
                                 (o_hbm, ov), (ar_hbm, hv))]
        for h in loads:
            h.wait()
        for t in range(B_PER_W // LANES):
            s = pl.ds(t * LANES, LANES)
            gaov.at[s][...] = (gaov.at[s][...] * 7 + tv.at[s][...]) * 21 + \
                ov.at[s][...]
            ar16 = hv.at[s][...]
            tv.at[s][...] = lax.shift_right_logical(ar16, 1)
            hv.at[s][...] = lax.shift_left(jnp.bitwise_and(ar16, 1), 6)

        def issue_gathers(c):
            s = c % 2
            off = pl.ds(c * CHUNK, CHUNK)
            return [
                pltpu.async_copy(wgao_hbm.at[gaov.at[off]], cats[s], semg[s]),
                pltpu.async_copy(xar_hbm.at[tv.at[off]], areas[s], semg[s]),
            ]

        def select_half(c):
            s = c % 2
            @pl.loop(0, CHUNK, step=LANES)
            def _(j0):
                rowv = iot + j0
                hvv = hv.at[pl.ds(c * CHUNK + j0, LANES)][...]
                colv = iot * 0 + (3 * EMBED_DIM)
                for cc in range(EMBED_DIM):
                    vals = plsc.load_gather(areas[s], [rowv, hvv + cc])
                    plsc.store_scatter(cats[s], [rowv, colv + cc], vals)

        pend_g = {0: issue_gathers(0), 1: None}
        pend_s = {0: None, 1: None}
        for c in range(NCH):
            s = c % 2
            for h in pend_g[s]:
                h.wait()
            if c + 1 < NCH:
                if pend_s[1 - s] is not None:
                    pend_s[1 - s].wait()
                    pend_s[1 - s] = None
                pend_g[1 - s] = issue_gathers(c + 1)
            select_half(c)
            pend_s[s] = pltpu.async_copy(
                cats[s], out_hbm.at[pl.ds(base + c * CHUNK, CHUNK)], sems[s])
        for s in (0, 1):
            if pend_s[s] is not None:
                pend_s[s].wait()

    return emb_kernel(g, a, o, ar, W_gao, X_area)
